# baseline (device time: 658448 ns/iter reference)
import jax
import jax.numpy as jnp
from jax import lax
from jax.experimental import pallas as pl
from jax.experimental.pallas import tpu as pltpu

N_DEV = 16
P = 4
NPASS = 8


def kernel(x, w_mat):
    m, k_per = x.shape
    _, n = w_mat.shape
    m_per = m // N_DEV
    PW = n // NPASS
    DW = PW // 2
    GR = P * m_per

    def body(x_ref, w_ref, out_ref, comm1A, comm1B, finA, finB,
             comm2A, comm2B,
             s1A_snd, s1A_rcv, s1B_snd, s1B_rcv,
             s2A_snd, s2A_rcv, s2B_snd, s2B_rcv):
        my = lax.axis_index("i")
        j = lax.rem(my, P)
        z = my // P
        plane_r = z * P + lax.rem(j + 1, P)
        plane_l = z * P + lax.rem(j + P - 1, P)
        z_up = lax.rem(my + P, N_DEV)
        z_dn = lax.rem(my + N_DEV - P, N_DEV)

        barrier_sem = pltpu.get_barrier_semaphore()
        for nbr in (plane_l, plane_r, z_dn, z_up):
            pl.semaphore_signal(
                barrier_sem, inc=1,
                device_id=(nbr,), device_id_type=pl.DeviceIdType.MESH,
            )
        pl.semaphore_wait(barrier_sem, 4)

        dirs = (
            (comm1A, finA, comm2A, s1A_snd, s1A_rcv, s2A_snd, s2A_rcv,
             plane_r, z_up),
            (comm1B, finB, comm2B, s1B_snd, s1B_rcv, s2B_snd, s2B_rcv,
             plane_l, z_dn),
        )

        def col0(d, p):
            return p * PW + d * DW

        def g_send(d, h):
            return lax.rem(j + P - 1 - h, P) if d == 0 else lax.rem(j + 1 + h, P)

        def g_recv(d, h):
            return lax.rem(j + P - 2 - h, P) if d == 0 else lax.rem(j + 2 + h, P)

        def p_send(d, k):
            return lax.rem(z + P - 1 - k, P) if d == 0 else lax.rem(z + 1 + k, P)

        def gelu(y):
            cg = 0.7978845608028654
            return 0.5 * y * (1.0 + jnp.tanh(cg * (y + 0.044715 * y * y * y)))

        def piece_gemm(d, g, zp, p):
            c0 = col0(d, p)
            return jnp.dot(
                x_ref[pl.ds((zp * P + g) * m_per, m_per), :],
                w_ref[:, c0:c0 + DW],
                preferred_element_type=jnp.float32,
            )

        desc1 = {}
        desc2 = {}

        def start1(d, h, p):
            comm1, fin = dirs[d][0], dirs[d][1]
            snd, rcv, tgt = dirs[d][3], dirs[d][4], dirs[d][7]
            if p >= 1 and h < 2:
                desc1[(d, h, p - 1)].wait_send()
            src = comm1.at[h % 2]
            dst = fin.at[p % 2] if h == 2 else comm1.at[(h + 1) % 2]
            dd = pltpu.make_async_remote_copy(
                src_ref=src, dst_ref=dst,
                send_sem=snd.at[h], recv_sem=rcv.at[h],
                device_id=(tgt,), device_id_type=pl.DeviceIdType.MESH,
            )
            desc1[(d, h, p)] = dd
            dd.start()

        def start2(d, k, p):
            fin, comm2 = dirs[d][1], dirs[d][2]
            snd, rcv, tgt = dirs[d][5], dirs[d][6], dirs[d][8]
            if p >= 1:
                desc2[(d, k, p - 1)].wait_send()
            if k == 0:
                src = fin.at[p % 2, pl.ds(p_send(d, 0) * m_per, m_per), :]
            else:
                src = comm2.at[k - 1]
            dd = pltpu.make_async_remote_copy(
                src_ref=src, dst_ref=comm2.at[k],
                send_sem=snd.at[k], recv_sem=rcv.at[k],
                device_id=(tgt,), device_id_type=pl.DeviceIdType.MESH,
            )
            desc2[(d, k, p)] = dd
            dd.start()

        def fill(d, p):
            comm1 = dirs[d][0]
            if p >= 1:
                desc1[(d, 2, p - 1)].wait_send()
            g = g_send(d, 0)
            for zp in range(P):
                comm1[0, zp * m_per:(zp + 1) * m_per, :] = \
                    piece_gemm(d, g, zp, p)

        def adv2(k, p):
            for d in range(2):
                fin, comm2 = dirs[d][1], dirs[d][2]
                desc2[(d, k - 1, p)].wait_recv()
                pc = p_send(d, k)
                comm2[k - 1, :, :] = (
                    comm2[k - 1, :, :]
                    + fin[p % 2, pl.ds(pc * m_per, m_per), :]
                )
                start2(d, k, p)

        def finalize(p):
            for d in range(2):
                fin, comm2 = dirs[d][1], dirs[d][2]
                desc2[(d, 2, p)].wait_recv()
                yv = comm2[2, :, :] + fin[p % 2, pl.ds(z * m_per, m_per), :]
                c0 = col0(d, p)
                out_ref[:, c0:c0 + DW] = gelu(yv)

        for d in range(2):
            fill(d, 0)
            start1(d, 0, 0)

        for p in range(NPASS):
            for h in range(3):
                vals = [[piece_gemm(d, g_recv(d, h), zp, p) for zp in range(P)]
                        for d in range(2)]
                if h == 1 and p >= 1:
                    adv2(1, p - 1)
                if h == 2 and p >= 1:
                    adv2(2, p - 1)
                for d in range(2):
                    comm1, fin = dirs[d][0], dirs[d][1]
                    desc1[(d, h, p)].wait_recv()
                    tgt_ref = fin if h == 2 else comm1
                    ts = p % 2 if h == 2 else (h + 1) % 2
                    for zp in range(P):
                        tgt_ref[ts, zp * m_per:(zp + 1) * m_per, :] = (
                            tgt_ref[ts, zp * m_per:(zp + 1) * m_per, :]
                            + vals[d][zp]
                        )
                    if h < 2:
                        start1(d, h + 1, p)
                    else:
                        start2(d, 0, p)
            if p >= 1:
                finalize(p - 1)
            if p + 1 < NPASS:
                for d in range(2):
                    fill(d, p + 1)
                    start1(d, 0, p + 1)

        adv2(1, NPASS - 1)
        adv2(2, NPASS - 1)
        finalize(NPASS - 1)

        for d in range(2):
            for h in range(3):
                desc1[(d, h, NPASS - 1)].wait_send()
            for k in range(3):
                desc2[(d, k, NPASS - 1)].wait_send()

    return pl.pallas_call(
        body,
        out_shape=jax.ShapeDtypeStruct((m_per, n), jnp.float32),
        in_specs=[
            pl.BlockSpec(memory_space=pltpu.VMEM),
            pl.BlockSpec(memory_space=pltpu.VMEM),
        ],
        out_specs=pl.BlockSpec(memory_space=pltpu.VMEM),
        scratch_shapes=[
            pltpu.VMEM((2, GR, DW), jnp.float32),
            pltpu.VMEM((2, GR, DW), jnp.float32),
            pltpu.VMEM((2, GR, DW), jnp.float32),
            pltpu.VMEM((2, GR, DW), jnp.float32),
            pltpu.VMEM((3, m_per, DW), jnp.float32),
            pltpu.VMEM((3, m_per, DW), jnp.float32),
            pltpu.SemaphoreType.DMA((3,)),
            pltpu.SemaphoreType.DMA((3,)),
            pltpu.SemaphoreType.DMA((3,)),
            pltpu.SemaphoreType.DMA((3,)),
            pltpu.SemaphoreType.DMA((3,)),
            pltpu.SemaphoreType.DMA((3,)),
            pltpu.SemaphoreType.DMA((3,)),
            pltpu.SemaphoreType.DMA((3,)),
        ],
        compiler_params=pltpu.CompilerParams(
            collective_id=0, vmem_limit_bytes=63 * 1024 * 1024
        ),
    )(x, w_mat)


# device time: 626820 ns/iter; 1.0505x vs baseline; 1.0505x over previous
import jax
import jax.numpy as jnp
from jax import lax
from jax.experimental import pallas as pl
from jax.experimental.pallas import tpu as pltpu

N_DEV = 16
P = 4
NPASS = 8


def kernel(x, w_mat):
    m, k_per = x.shape
    _, n = w_mat.shape
    m_per = m // N_DEV
    PW = n // NPASS
    DW = PW // 2
    SUBW = DW // 2
    GR = P * m_per

    def body(x_ref, w_ref, out_ref, comm1A, comm1B, finA, finB,
             comm2A, comm2B,
             s1A_snd, s1A_rcv, s1B_snd, s1B_rcv,
             s2A_snd, s2A_rcv, s2B_snd, s2B_rcv):
        my = lax.axis_index("i")
        j = lax.rem(my, P)
        z = my // P
        plane_r = z * P + lax.rem(j + 1, P)
        plane_l = z * P + lax.rem(j + P - 1, P)
        z_up = lax.rem(my + P, N_DEV)
        z_dn = lax.rem(my + N_DEV - P, N_DEV)

        barrier_sem = pltpu.get_barrier_semaphore()
        for nbr in (plane_l, plane_r, z_dn, z_up):
            pl.semaphore_signal(
                barrier_sem, inc=1,
                device_id=(nbr,), device_id_type=pl.DeviceIdType.MESH,
            )
        pl.semaphore_wait(barrier_sem, 4)

        dirs = (
            (comm1A, finA, comm2A, s1A_snd, s1A_rcv, s2A_snd, s2A_rcv,
             plane_r, z_up),
            (comm1B, finB, comm2B, s1B_snd, s1B_rcv, s2B_snd, s2B_rcv,
             plane_l, z_dn),
        )

        def col0(d, p):
            return p * PW + d * DW

        def g_send(d, h):
            return lax.rem(j + P - 1 - h, P) if d == 0 else lax.rem(j + 1 + h, P)

        def g_recv(d, h):
            return lax.rem(j + P - 2 - h, P) if d == 0 else lax.rem(j + 2 + h, P)

        def p_send(d, k):
            return lax.rem(z + P - 1 - k, P) if d == 0 else lax.rem(z + 1 + k, P)

        def gelu(y):
            cg = 0.7978845608028654
            return 0.5 * y * (1.0 + jnp.tanh(cg * (y + 0.044715 * y * y * y)))

        def piece_gemm(d, g, zp, p, sub):
            c0 = col0(d, p) + sub * SUBW
            return jnp.dot(
                x_ref[pl.ds((zp * P + g) * m_per, m_per), :],
                w_ref[:, c0:c0 + SUBW],
                preferred_element_type=jnp.float32,
            )

        desc1 = {}
        desc2 = {}

        def start1(d, h, p, sub):
            comm1, fin = dirs[d][0], dirs[d][1]
            snd, rcv, tgt = dirs[d][3], dirs[d][4], dirs[d][7]
            if p >= 1 and h < 2:
                desc1[(d, h, p - 1, sub)].wait_send()
            cs = slice(sub * SUBW, (sub + 1) * SUBW)
            src = comm1.at[h % 2, :, cs]
            dst = fin.at[p % 2, :, cs] if h == 2 else comm1.at[(h + 1) % 2, :, cs]
            dd = pltpu.make_async_remote_copy(
                src_ref=src, dst_ref=dst,
                send_sem=snd.at[h, sub], recv_sem=rcv.at[h, sub],
                device_id=(tgt,), device_id_type=pl.DeviceIdType.MESH,
            )
            desc1[(d, h, p, sub)] = dd
            dd.start()

        def start2(d, k, p):
            fin, comm2 = dirs[d][1], dirs[d][2]
            snd, rcv, tgt = dirs[d][5], dirs[d][6], dirs[d][8]
            if p >= 1:
                desc2[(d, k, p - 1)].wait_send()
            if k == 0:
                src = fin.at[p % 2, pl.ds(p_send(d, 0) * m_per, m_per), :]
            else:
                src = comm2.at[k - 1]
            dd = pltpu.make_async_remote_copy(
                src_ref=src, dst_ref=comm2.at[k],
                send_sem=snd.at[k], recv_sem=rcv.at[k],
                device_id=(tgt,), device_id_type=pl.DeviceIdType.MESH,
            )
            desc2[(d, k, p)] = dd
            dd.start()

        def fill(d, p, sub):
            comm1 = dirs[d][0]
            if p >= 1:
                desc1[(d, 2, p - 1, sub)].wait_send()
            g = g_send(d, 0)
            cs = slice(sub * SUBW, (sub + 1) * SUBW)
            for zp in range(P):
                comm1[0, zp * m_per:(zp + 1) * m_per, cs] = \
                    piece_gemm(d, g, zp, p, sub)

        def adv2(k, p):
            for d in range(2):
                fin, comm2 = dirs[d][1], dirs[d][2]
                desc2[(d, k - 1, p)].wait_recv()
                pc = p_send(d, k)
                comm2[k - 1, :, :] = (
                    comm2[k - 1, :, :]
                    + fin[p % 2, pl.ds(pc * m_per, m_per), :]
                )
                start2(d, k, p)

        def finalize(p):
            for d in range(2):
                fin, comm2 = dirs[d][1], dirs[d][2]
                desc2[(d, 2, p)].wait_recv()
                yv = comm2[2, :, :] + fin[p % 2, pl.ds(z * m_per, m_per), :]
                c0 = col0(d, p)
                out_ref[:, c0:c0 + DW] = gelu(yv)

        for sub in range(2):
            for d in range(2):
                fill(d, 0, sub)
                start1(d, 0, 0, sub)

        for p in range(NPASS):
            for h in range(3):
                if h == 1 and p >= 1:
                    adv2(1, p - 1)
                if h == 2 and p >= 1:
                    adv2(2, p - 1)
                for sub in range(2):
                    vals = [[piece_gemm(d, g_recv(d, h), zp, p, sub)
                             for zp in range(P)] for d in range(2)]
                    cs = slice(sub * SUBW, (sub + 1) * SUBW)
                    for d in range(2):
                        comm1, fin = dirs[d][0], dirs[d][1]
                        desc1[(d, h, p, sub)].wait_recv()
                        tgt_ref = fin if h == 2 else comm1
                        ts = p % 2 if h == 2 else (h + 1) % 2
                        for zp in range(P):
                            tgt_ref[ts, zp * m_per:(zp + 1) * m_per, cs] = (
                                tgt_ref[ts, zp * m_per:(zp + 1) * m_per, cs]
                                + vals[d][zp]
                            )
                        if h < 2:
                            start1(d, h + 1, p, sub)
                        elif sub == 1:
                            start2(d, 0, p)
            if p >= 1:
                finalize(p - 1)
            if p + 1 < NPASS:
                for sub in range(2):
                    for d in range(2):
                        fill(d, p + 1, sub)
                        start1(d, 0, p + 1, sub)

        adv2(1, NPASS - 1)
        adv2(2, NPASS - 1)
        finalize(NPASS - 1)

        for d in range(2):
            for h in range(3):
                for sub in range(2):
                    desc1[(d, h, NPASS - 1, sub)].wait_send()
            for k in range(3):
                desc2[(d, k, NPASS - 1)].wait_send()

    return pl.pallas_call(
        body,
        out_shape=jax.ShapeDtypeStruct((m_per, n), jnp.float32),
        in_specs=[
            pl.BlockSpec(memory_space=pltpu.VMEM),
            pl.BlockSpec(memory_space=pltpu.VMEM),
        ],
        out_specs=pl.BlockSpec(memory_space=pltpu.VMEM),
        scratch_shapes=[
            pltpu.VMEM((2, GR, DW), jnp.float32),
            pltpu.VMEM((2, GR, DW), jnp.float32),
            pltpu.VMEM((2, GR, DW), jnp.float32),
            pltpu.VMEM((2, GR, DW), jnp.float32),
            pltpu.VMEM((3, m_per, DW), jnp.float32),
            pltpu.VMEM((3, m_per, DW), jnp.float32),
            pltpu.SemaphoreType.DMA((3, 2)),
            pltpu.SemaphoreType.DMA((3, 2)),
            pltpu.SemaphoreType.DMA((3, 2)),
            pltpu.SemaphoreType.DMA((3, 2)),
            pltpu.SemaphoreType.DMA((3,)),
            pltpu.SemaphoreType.DMA((3,)),
            pltpu.SemaphoreType.DMA((3,)),
            pltpu.SemaphoreType.DMA((3,)),
        ],
        compiler_params=pltpu.CompilerParams(
            collective_id=0, vmem_limit_bytes=63 * 1024 * 1024
        ),
    )(x, w_mat)


# device time: 601002 ns/iter; 1.0956x vs baseline; 1.0430x over previous
import jax
import jax.numpy as jnp
from jax import lax
from jax.experimental import pallas as pl
from jax.experimental.pallas import tpu as pltpu

N_DEV = 16
P = 4
NPASS = 8


def kernel(x, w_mat):
    m, k_per = x.shape
    _, n = w_mat.shape
    m_per = m // N_DEV
    PW = n // NPASS
    DW = PW // 2
    SUBW = DW // 2
    GR = P * m_per

    def body(x_ref, w_ref, out_ref, comm1A, comm1B, finA, finB,
             comm2A, comm2B,
             s1A_snd, s1A_rcv, s1B_snd, s1B_rcv,
             s2A_snd, s2A_rcv, s2B_snd, s2B_rcv):
        my = lax.axis_index("i")
        j = lax.rem(my, P)
        z = my // P
        plane_r = z * P + lax.rem(j + 1, P)
        plane_l = z * P + lax.rem(j + P - 1, P)
        z_up = lax.rem(my + P, N_DEV)
        z_dn = lax.rem(my + N_DEV - P, N_DEV)

        barrier_sem = pltpu.get_barrier_semaphore()
        for nbr in (plane_l, plane_r, z_dn, z_up):
            pl.semaphore_signal(
                barrier_sem, inc=1,
                device_id=(nbr,), device_id_type=pl.DeviceIdType.MESH,
            )
        pl.semaphore_wait(barrier_sem, 4)

        dirs = (
            (comm1A, finA, comm2A, s1A_snd, s1A_rcv, s2A_snd, s2A_rcv,
             plane_r, z_up),
            (comm1B, finB, comm2B, s1B_snd, s1B_rcv, s2B_snd, s2B_rcv,
             plane_l, z_dn),
        )

        def col0(d, p):
            return p * PW + d * DW

        def g_send(d, h):
            return lax.rem(j + P - 1 - h, P) if d == 0 else lax.rem(j + 1 + h, P)

        def g_recv(d, h):
            return lax.rem(j + P - 2 - h, P) if d == 0 else lax.rem(j + 2 + h, P)

        def p_send(d, k):
            return lax.rem(z + P - 1 - k, P) if d == 0 else lax.rem(z + 1 + k, P)

        def gelu(y):
            cg = 0.7978845608028654
            return 0.5 * y * (1.0 + jnp.tanh(cg * (y + 0.044715 * y * y * y)))

        def piece_gemm(d, g, zp, p, sub):
            c0 = col0(d, p) + sub * SUBW
            return jnp.dot(
                x_ref[pl.ds((zp * P + g) * m_per, m_per), :],
                w_ref[:, c0:c0 + SUBW],
                preferred_element_type=jnp.float32,
            )

        desc1 = {}
        desc2 = {}

        def start1(d, h, p, sub):
            comm1, fin = dirs[d][0], dirs[d][1]
            snd, rcv, tgt = dirs[d][3], dirs[d][4], dirs[d][7]
            if p >= 1 and h < 2:
                desc1[(d, h, p - 1, sub)].wait_send()
            cs = slice(sub * SUBW, (sub + 1) * SUBW)
            src = comm1.at[h % 2, :, cs]
            dst = fin.at[p % 2, :, cs] if h == 2 else comm1.at[(h + 1) % 2, :, cs]
            dd = pltpu.make_async_remote_copy(
                src_ref=src, dst_ref=dst,
                send_sem=snd.at[h, sub], recv_sem=rcv.at[h, sub],
                device_id=(tgt,), device_id_type=pl.DeviceIdType.MESH,
            )
            desc1[(d, h, p, sub)] = dd
            dd.start()

        def start2(d, k, p):
            fin, comm2 = dirs[d][1], dirs[d][2]
            snd, rcv, tgt = dirs[d][5], dirs[d][6], dirs[d][8]
            if p >= 1:
                desc2[(d, k, p - 1)].wait_send()
            if k == 0:
                src = fin.at[p % 2, pl.ds(p_send(d, 0) * m_per, m_per), :]
            else:
                src = comm2.at[k - 1]
            dd = pltpu.make_async_remote_copy(
                src_ref=src, dst_ref=comm2.at[k],
                send_sem=snd.at[k], recv_sem=rcv.at[k],
                device_id=(tgt,), device_id_type=pl.DeviceIdType.MESH,
            )
            desc2[(d, k, p)] = dd
            dd.start()

        def fill(d, p, sub):
            comm1 = dirs[d][0]
            if p >= 1:
                desc1[(d, 2, p - 1, sub)].wait_send()
            g = g_send(d, 0)
            cs = slice(sub * SUBW, (sub + 1) * SUBW)
            for zp in range(P):
                comm1[0, zp * m_per:(zp + 1) * m_per, cs] = \
                    piece_gemm(d, g, zp, p, sub)

        def adv2(k, p):
            for d in range(2):
                fin, comm2 = dirs[d][1], dirs[d][2]
                desc2[(d, k - 1, p)].wait_recv()
                pc = p_send(d, k)
                comm2[k - 1, :, :] = (
                    comm2[k - 1, :, :]
                    + fin[p % 2, pl.ds(pc * m_per, m_per), :]
                )
                start2(d, k, p)

        def finalize(p):
            for d in range(2):
                fin, comm2 = dirs[d][1], dirs[d][2]
                desc2[(d, 2, p)].wait_recv()
                yv = comm2[2, :, :] + fin[p % 2, pl.ds(z * m_per, m_per), :]
                c0 = col0(d, p)
                out_ref[:, c0:c0 + DW] = gelu(yv)

        for sub in range(2):
            for d in range(2):
                fill(d, 0, sub)
                start1(d, 0, 0, sub)

        for p in range(NPASS):
            for h in range(3):
                if h == 1 and p >= 1:
                    adv2(1, p - 1)
                if h == 2 and p >= 1:
                    adv2(2, p - 1)
                for sub in range(2):
                    vals = [[piece_gemm(d, g_recv(d, h), zp, p, sub)
                             for zp in range(P)] for d in range(2)]
                    cs = slice(sub * SUBW, (sub + 1) * SUBW)
                    for d in range(2):
                        comm1, fin = dirs[d][0], dirs[d][1]
                        desc1[(d, h, p, sub)].wait_recv()
                        tgt_ref = fin if h == 2 else comm1
                        ts = p % 2 if h == 2 else (h + 1) % 2
                        for zp in range(P):
                            tgt_ref[ts, zp * m_per:(zp + 1) * m_per, cs] = (
                                tgt_ref[ts, zp * m_per:(zp + 1) * m_per, cs]
                                + vals[d][zp]
                            )
                        if h < 2:
                            start1(d, h + 1, p, sub)
                        elif sub == 1:
                            start2(d, 0, p)
                    if h == 2 and p + 1 < NPASS:
                        for d in range(2):
                            fill(d, p + 1, sub)
                            start1(d, 0, p + 1, sub)
            if p >= 1:
                finalize(p - 1)

        adv2(1, NPASS - 1)
        adv2(2, NPASS - 1)
        finalize(NPASS - 1)

        for d in range(2):
            for h in range(3):
                for sub in range(2):
                    desc1[(d, h, NPASS - 1, sub)].wait_send()
            for k in range(3):
                desc2[(d, k, NPASS - 1)].wait_send()

    return pl.pallas_call(
        body,
        out_shape=jax.ShapeDtypeStruct((m_per, n), jnp.float32),
        in_specs=[
            pl.BlockSpec(memory_space=pltpu.VMEM),
            pl.BlockSpec(memory_space=pltpu.VMEM),
        ],
        out_specs=pl.BlockSpec(memory_space=pltpu.VMEM),
        scratch_shapes=[
            pltpu.VMEM((2, GR, DW), jnp.float32),
            pltpu.VMEM((2, GR, DW), jnp.float32),
            pltpu.VMEM((2, GR, DW), jnp.float32),
            pltpu.VMEM((2, GR, DW), jnp.float32),
            pltpu.VMEM((3, m_per, DW), jnp.float32),
            pltpu.VMEM((3, m_per, DW), jnp.float32),
            pltpu.SemaphoreType.DMA((3, 2)),
            pltpu.SemaphoreType.DMA((3, 2)),
            pltpu.SemaphoreType.DMA((3, 2)),
            pltpu.SemaphoreType.DMA((3, 2)),
            pltpu.SemaphoreType.DMA((3,)),
            pltpu.SemaphoreType.DMA((3,)),
            pltpu.SemaphoreType.DMA((3,)),
            pltpu.SemaphoreType.DMA((3,)),
        ],
        compiler_params=pltpu.CompilerParams(
            collective_id=0, vmem_limit_bytes=63 * 1024 * 1024
        ),
    )(x, w_mat)


# device time: 594241 ns/iter; 1.1080x vs baseline; 1.0114x over previous
import jax
import jax.numpy as jnp
from jax import lax
from jax.experimental import pallas as pl
from jax.experimental.pallas import tpu as pltpu

N_DEV = 16
P = 4
NPASS = 8


def kernel(x, w_mat):
    m, k_per = x.shape
    _, n = w_mat.shape
    m_per = m // N_DEV
    PW = n // NPASS
    DW = PW // 2
    SUBW = DW // 2
    GR = P * m_per

    def body(x_ref, w_ref, out_ref, comm1A, comm1B, finA, finB,
             comm2A, comm2B,
             s1A_snd, s1A_rcv, s1B_snd, s1B_rcv,
             s2A_snd, s2A_rcv, s2B_snd, s2B_rcv):
        my = lax.axis_index("i")
        j = lax.rem(my, P)
        z = my // P
        plane_r = z * P + lax.rem(j + 1, P)
        plane_l = z * P + lax.rem(j + P - 1, P)
        z_up = lax.rem(my + P, N_DEV)
        z_dn = lax.rem(my + N_DEV - P, N_DEV)

        barrier_sem = pltpu.get_barrier_semaphore()
        for nbr in (plane_l, plane_r, z_dn, z_up):
            pl.semaphore_signal(
                barrier_sem, inc=1,
                device_id=(nbr,), device_id_type=pl.DeviceIdType.MESH,
            )
        pl.semaphore_wait(barrier_sem, 4)

        dirs = (
            (comm1A, finA, comm2A, s1A_snd, s1A_rcv, s2A_snd, s2A_rcv,
             plane_r, z_up),
            (comm1B, finB, comm2B, s1B_snd, s1B_rcv, s2B_snd, s2B_rcv,
             plane_l, z_dn),
        )

        def col0(d, p):
            return p * PW + d * DW

        def g_send(d, h):
            return lax.rem(j + P - 1 - h, P) if d == 0 else lax.rem(j + 1 + h, P)

        def g_recv(d, h):
            return lax.rem(j + P - 2 - h, P) if d == 0 else lax.rem(j + 2 + h, P)

        def p_send(d, k):
            return lax.rem(z + P - 1 - k, P) if d == 0 else lax.rem(z + 1 + k, P)

        def gelu(y):
            cg = 0.7978845608028654
            return 0.5 * y * (1.0 + jnp.tanh(cg * (y + 0.044715 * y * y * y)))

        def piece_gemm(d, g, zp, p, sub):
            c0 = col0(d, p) + sub * SUBW
            return jnp.dot(
                x_ref[pl.ds((zp * P + g) * m_per, m_per), :],
                w_ref[:, c0:c0 + SUBW],
                preferred_element_type=jnp.float32,
            )

        desc1 = {}
        desc2 = {}

        def start1(d, h, p, sub):
            comm1, fin = dirs[d][0], dirs[d][1]
            snd, rcv, tgt = dirs[d][3], dirs[d][4], dirs[d][7]
            if p >= 1 and h < 2:
                desc1[(d, h, p - 1, sub)].wait_send()
            cs = slice(sub * SUBW, (sub + 1) * SUBW)
            src = comm1.at[h % 2, :, cs]
            dst = fin.at[p % 2, :, cs] if h == 2 else comm1.at[(h + 1) % 2, :, cs]
            dd = pltpu.make_async_remote_copy(
                src_ref=src, dst_ref=dst,
                send_sem=snd.at[h, sub], recv_sem=rcv.at[h, sub],
                device_id=(tgt,), device_id_type=pl.DeviceIdType.MESH,
            )
            desc1[(d, h, p, sub)] = dd
            dd.start()

        def start2(d, k, p, sub):
            fin, comm2 = dirs[d][1], dirs[d][2]
            snd, rcv, tgt = dirs[d][5], dirs[d][6], dirs[d][8]
            if p >= 1:
                desc2[(d, k, p - 1, sub)].wait_send()
            cs = slice(sub * SUBW, (sub + 1) * SUBW)
            if k == 0:
                src = fin.at[p % 2, pl.ds(p_send(d, 0) * m_per, m_per), cs]
            else:
                src = comm2.at[k - 1, :, cs]
            dd = pltpu.make_async_remote_copy(
                src_ref=src, dst_ref=comm2.at[k, :, cs],
                send_sem=snd.at[k, sub], recv_sem=rcv.at[k, sub],
                device_id=(tgt,), device_id_type=pl.DeviceIdType.MESH,
            )
            desc2[(d, k, p, sub)] = dd
            dd.start()

        def fill(d, p, sub):
            comm1 = dirs[d][0]
            if p >= 1:
                desc1[(d, 2, p - 1, sub)].wait_send()
            g = g_send(d, 0)
            cs = slice(sub * SUBW, (sub + 1) * SUBW)
            for zp in range(P):
                comm1[0, zp * m_per:(zp + 1) * m_per, cs] = \
                    piece_gemm(d, g, zp, p, sub)

        def adv2(k, p):
            for sub in range(2):
                cs = slice(sub * SUBW, (sub + 1) * SUBW)
                for d in range(2):
                    fin, comm2 = dirs[d][1], dirs[d][2]
                    desc2[(d, k - 1, p, sub)].wait_recv()
                    pc = p_send(d, k)
                    comm2[k - 1, :, cs] = (
                        comm2[k - 1, :, cs]
                        + fin[p % 2, pl.ds(pc * m_per, m_per), cs]
                    )
                    start2(d, k, p, sub)

        def finalize(p):
            for sub in range(2):
                cs = slice(sub * SUBW, (sub + 1) * SUBW)
                for d in range(2):
                    fin, comm2 = dirs[d][1], dirs[d][2]
                    desc2[(d, 2, p, sub)].wait_recv()
                    yv = (comm2[2, :, cs]
                          + fin[p % 2, pl.ds(z * m_per, m_per), cs])
                    c0 = col0(d, p) + sub * SUBW
                    out_ref[:, c0:c0 + SUBW] = gelu(yv)

        for sub in range(2):
            for d in range(2):
                fill(d, 0, sub)
                start1(d, 0, 0, sub)

        for p in range(NPASS):
            for h in range(3):
                if h == 1 and p >= 1:
                    adv2(1, p - 1)
                if h == 2 and p >= 1:
                    adv2(2, p - 1)
                for sub in range(2):
                    vals = [[piece_gemm(d, g_recv(d, h), zp, p, sub)
                             for zp in range(P)] for d in range(2)]
                    cs = slice(sub * SUBW, (sub + 1) * SUBW)
                    for d in range(2):
                        comm1, fin = dirs[d][0], dirs[d][1]
                        desc1[(d, h, p, sub)].wait_recv()
                        tgt_ref = fin if h == 2 else comm1
                        ts = p % 2 if h == 2 else (h + 1) % 2
                        for zp in range(P):
                            tgt_ref[ts, zp * m_per:(zp + 1) * m_per, cs] = (
                                tgt_ref[ts, zp * m_per:(zp + 1) * m_per, cs]
                                + vals[d][zp]
                            )
                        if h < 2:
                            start1(d, h + 1, p, sub)
                        else:
                            start2(d, 0, p, sub)
                    if h == 2 and p + 1 < NPASS:
                        for d in range(2):
                            fill(d, p + 1, sub)
                            start1(d, 0, p + 1, sub)
            if p >= 1:
                finalize(p - 1)

        adv2(1, NPASS - 1)
        adv2(2, NPASS - 1)
        finalize(NPASS - 1)

        for d in range(2):
            for h in range(3):
                for sub in range(2):
                    desc1[(d, h, NPASS - 1, sub)].wait_send()
            for k in range(3):
                for sub in range(2):
                    desc2[(d, k, NPASS - 1, sub)].wait_send()

    return pl.pallas_call(
        body,
        out_shape=jax.ShapeDtypeStruct((m_per, n), jnp.float32),
        in_specs=[
            pl.BlockSpec(memory_space=pltpu.VMEM),
            pl.BlockSpec(memory_space=pltpu.VMEM),
        ],
        out_specs=pl.BlockSpec(memory_space=pltpu.VMEM),
        scratch_shapes=[
            pltpu.VMEM((2, GR, DW), jnp.float32),
            pltpu.VMEM((2, GR, DW), jnp.float32),
            pltpu.VMEM((2, GR, DW), jnp.float32),
            pltpu.VMEM((2, GR, DW), jnp.float32),
            pltpu.VMEM((3, m_per, DW), jnp.float32),
            pltpu.VMEM((3, m_per, DW), jnp.float32),
            pltpu.SemaphoreType.DMA((3, 2)),
            pltpu.SemaphoreType.DMA((3, 2)),
            pltpu.SemaphoreType.DMA((3, 2)),
            pltpu.SemaphoreType.DMA((3, 2)),
            pltpu.SemaphoreType.DMA((3, 2)),
            pltpu.SemaphoreType.DMA((3, 2)),
            pltpu.SemaphoreType.DMA((3, 2)),
            pltpu.SemaphoreType.DMA((3, 2)),
        ],
        compiler_params=pltpu.CompilerParams(
            collective_id=0, vmem_limit_bytes=63 * 1024 * 1024
        ),
    )(x, w_mat)
